# baseline (speedup 1.0000x reference)
import functools

import jax
import jax.numpy as jnp
from jax import lax
from jax.experimental import pallas as pl
from jax.experimental.pallas import tpu as pltpu
from jax.experimental.pallas import tpu_sc as plsc

BTM = 256
CH = 16
CHC = 8


def _routing_body(x_ref, gw_ref, w0_ref, w1_ref, p0_ref, p1_ref, bmap_ref,
                  cnt_ref, e0s_ref, e1s_ref, r0s_ref, r1s_ref):
    s = pl.program_id(0)
    NS = pl.num_programs(0)
    S = x_ref.shape[0]
    E = gw_ref.shape[0]
    NBLKP = bmap_ref.shape[1]
    T = e0s_ref.shape[0]

    @pl.when(s == 0)
    def _():
        cnt_ref[...] = jnp.zeros_like(cnt_ref)

    x = x_ref[...]
    logits = lax.dot_general(x, gw_ref[...], (((1,), (1,)), ((), ())),
                             preferred_element_type=jnp.float32)
    iota = lax.broadcasted_iota(jnp.int32, (S, E), 1)
    m = jnp.max(logits, axis=-1, keepdims=True)
    p = jnp.exp(logits - m)
    m1 = jnp.max(p, axis=-1, keepdims=True)
    i1 = jnp.min(jnp.where(p == m1, iota, E), axis=-1, keepdims=True)
    mask1 = iota == i1
    p2 = jnp.where(mask1, -jnp.inf, p)
    m2 = jnp.max(p2, axis=-1, keepdims=True)
    i2 = jnp.min(jnp.where(p2 == m2, iota, E), axis=-1, keepdims=True)
    mask2 = iota == i2
    denom = m1 + m2
    w0_ref[...] = jnp.broadcast_to(m1 / denom, w0_ref.shape)
    w1_ref[...] = jnp.broadcast_to(m2 / denom, w1_ref.shape)

    oh = mask1.astype(jnp.float32) + mask2.astype(jnp.float32)
    tril = (lax.broadcasted_iota(jnp.int32, (S, S), 0) >
            lax.broadcasted_iota(jnp.int32, (S, S), 1)).astype(jnp.float32)
    csum = lax.dot_general(tril, oh, (((1,), (0,)), ((), ())),
                           preferred_element_type=jnp.float32)
    csum = csum + cnt_ref[...].astype(jnp.float32)
    r0 = jnp.sum(jnp.where(mask1, csum, 0.0), axis=-1,
                 keepdims=True).astype(jnp.int32)
    r1 = jnp.sum(jnp.where(mask2, csum, 0.0), axis=-1,
                 keepdims=True).astype(jnp.int32)

    sl = pl.ds(s * S, S)
    e0s_ref[sl] = i1
    e1s_ref[sl] = i2
    r0s_ref[sl] = r0
    r1s_ref[sl] = r1
    cnt_ref[...] = (cnt_ref[...] +
                    jnp.sum(oh, axis=0, keepdims=True).astype(jnp.int32))

    @pl.when(s == NS - 1)
    def _():
        cnt = cnt_ref[...]
        padded = ((cnt + (BTM - 1)) // BTM) * BTM
        upper = (lax.broadcasted_iota(jnp.int32, (E, E), 0) <
                 lax.broadcasted_iota(jnp.int32, (E, E), 1)).astype(jnp.float32)
        offs = lax.dot_general(padded.astype(jnp.float32), upper,
                               (((1,), (0,)), ((), ())),
                               preferred_element_type=jnp.float32)
        iota_te = lax.broadcasted_iota(jnp.int32, (T, E), 1)
        mask0 = (iota_te == e0s_ref[...]).astype(jnp.float32)
        mask1a = (iota_te == e1s_ref[...]).astype(jnp.float32)
        base0 = lax.dot_general(mask0, offs, (((1,), (1,)), ((), ())),
                                preferred_element_type=jnp.float32)
        base1 = lax.dot_general(mask1a, offs, (((1,), (1,)), ((), ())),
                                preferred_element_type=jnp.float32)
        p0_ref[...] = base0.astype(jnp.int32) + r0s_ref[...]
        p1_ref[...] = base1.astype(jnp.int32) + r1s_ref[...]

        incl = offs + padded.astype(jnp.float32)
        total = jnp.sum(padded)
        lane = lax.broadcasted_iota(jnp.int32, (1, NBLKP), 1)
        pos = jnp.minimum(lane * BTM, total - BTM).astype(jnp.float32)
        bexp = jnp.zeros((1, NBLKP), jnp.int32)
        for e in range(E):
            bexp = bexp + (pos >= incl[:, e:e + 1]).astype(jnp.int32)
        nblk_used = (total // BTM).astype(jnp.int32)
        bmap_ref[...] = jnp.where(lane == NBLKP - 1, nblk_used, bexp)


def _moe_body(bmap_ref, xs_ref, w1_ref, w3_ref, w2_ref, ws_ref, o_ref):
    b = pl.program_id(0)
    nused = bmap_ref[bmap_ref.shape[0] - 1]

    @pl.when(b < nused)
    def _():
        x = xs_ref[...]
        g = lax.dot_general(x, w1_ref[0], (((1,), (1,)), ((), ())),
                            preferred_element_type=jnp.float32)
        u = lax.dot_general(x, w3_ref[0], (((1,), (1,)), ((), ())),
                            preferred_element_type=jnp.float32)
        h = g * lax.logistic(g) * u
        y = lax.dot_general(h, w2_ref[0], (((1,), (1,)), ((), ())),
                            preferred_element_type=jnp.float32)
        o_ref[...] = y * ws_ref[:, 0:1]


def _sc_worker_id():
    return lax.axis_index("s") * 2 + lax.axis_index("c")


def _sc_dispatch_body(x_hbm, p0_hbm, p1_hbm, w0_hbm, w1_hbm, xs_hbm, ws_hbm,
                      rows_a, rows_b, p0_v, p1_v, w0_v, w1_v, lsem, ssem):
    T = x_hbm.shape[0]
    wid = _sc_worker_id()
    per_w = T // 32
    nch = per_w // CH
    base = wid * per_w
    pltpu.sync_copy(p0_hbm.at[pl.ds(base, per_w)], p0_v)
    pltpu.sync_copy(p1_hbm.at[pl.ds(base, per_w)], p1_v)
    pltpu.sync_copy(w0_hbm.at[pl.ds(base, per_w)], w0_v)
    pltpu.sync_copy(w1_hbm.at[pl.ds(base, per_w)], w1_v)
    bufs = [rows_a, rows_b]
    loads = {0: pltpu.async_copy(x_hbm.at[pl.ds(base, CH)], rows_a, lsem)}
    scats = {}
    for ci in range(nch):
        if ci >= 1:
            for c in scats[ci - 1]:
                c.wait()
        if ci + 1 < nch:
            loads[ci + 1] = pltpu.async_copy(
                x_hbm.at[pl.ds(base + (ci + 1) * CH, CH)],
                bufs[(ci + 1) % 2], lsem)
        loads[ci].wait()
        sl = pl.ds(ci * CH, CH)
        i0 = p0_v[sl]
        i1 = p1_v[sl]
        buf = bufs[ci % 2]
        scats[ci] = (pltpu.async_copy(buf, xs_hbm.at[i0], ssem),
                     pltpu.async_copy(buf, xs_hbm.at[i1], ssem),
                     pltpu.async_copy(w0_v.at[sl], ws_hbm.at[i0], ssem),
                     pltpu.async_copy(w1_v.at[sl], ws_hbm.at[i1], ssem))
    for c in scats[nch - 1]:
        c.wait()


def _sc_combine_body(ys_hbm, p0_hbm, p1_hbm, out_hbm, a0, a1, b0, b1,
                     oa, ob, p0_v, p1_v, gsem, ssem):
    T2 = out_hbm.shape[0]
    DH = out_hbm.shape[1]
    wid = _sc_worker_id()
    per_w = T2 // 32
    nch = per_w // 16
    base = wid * per_w
    pltpu.sync_copy(p0_hbm.at[pl.ds(base, per_w)], p0_v)
    pltpu.sync_copy(p1_hbm.at[pl.ds(base, per_w)], p1_v)

    rows = [(a0, a1), (b0, b1)]
    outs = [oa, ob]

    def gath(ci):
        r0, r1 = rows[ci % 2]
        return (pltpu.async_copy(ys_hbm.at[p0_v[pl.ds(ci * 16, 16)]], r0,
                                 gsem),
                pltpu.async_copy(ys_hbm.at[p1_v[pl.ds(ci * 16, 16)]], r1,
                                 gsem))

    gs = {0: gath(0)}
    st = {}
    for ci in range(nch):
        r0, r1 = rows[ci % 2]
        o = outs[ci % 2]
        if ci + 1 < nch:
            gs[ci + 1] = gath(ci + 1)
        gs[ci][0].wait()
        gs[ci][1].wait()
        if ci >= 2:
            st[ci - 2].wait()

        def body(j, _):
            s = pl.ds(j * 16, 16)
            for t in range(16):
                o[t, s] = r0[t, s] + r1[t, s]
            return 0

        lax.fori_loop(0, DH // 16, body, 0)
        st[ci] = pltpu.async_copy(o, out_hbm.at[pl.ds(base + ci * 16, 16)],
                                  ssem)
    st[nch - 2].wait()
    st[nch - 1].wait()


def kernel(hidden_states, gate_w, w1, w2, w3, num_global_tokens,
           max_num_tokens_per_gpu):
    T, D = hidden_states.shape
    E, FF, _ = w1.shape
    K = 2
    x = hidden_states.astype(jnp.float32)

    NBLK = (T * K) // BTM + E
    NBLKP = NBLK + 1
    NROWS = NBLK * BTM
    S = T // 2

    w0, w1c, p0, p1, bmap = pl.pallas_call(
        _routing_body,
        grid=(T // S,),
        in_specs=[
            pl.BlockSpec((S, D), lambda s: (s, 0)),
            pl.BlockSpec((E, D), lambda s: (0, 0)),
        ],
        out_specs=[
            pl.BlockSpec((S, 128), lambda s: (s, 0)),
            pl.BlockSpec((S, 128), lambda s: (s, 0)),
            pl.BlockSpec((T, 1), lambda s: (0, 0)),
            pl.BlockSpec((T, 1), lambda s: (0, 0)),
            pl.BlockSpec((1, NBLKP), lambda s: (0, 0)),
        ],
        out_shape=[
            jax.ShapeDtypeStruct((T, 128), jnp.float32),
            jax.ShapeDtypeStruct((T, 128), jnp.float32),
            jax.ShapeDtypeStruct((T, 1), jnp.int32),
            jax.ShapeDtypeStruct((T, 1), jnp.int32),
            jax.ShapeDtypeStruct((1, NBLKP), jnp.int32),
        ],
        scratch_shapes=[
            pltpu.VMEM((1, E), jnp.int32),
            pltpu.VMEM((T, 1), jnp.int32),
            pltpu.VMEM((T, 1), jnp.int32),
            pltpu.VMEM((T, 1), jnp.int32),
            pltpu.VMEM((T, 1), jnp.int32),
        ],
        compiler_params=pltpu.CompilerParams(
            dimension_semantics=("arbitrary",)),
    )(x, gate_w)

    p0f = p0.reshape(T)
    p1f = p1.reshape(T)
    PW = T // 32

    mesh = plsc.VectorSubcoreMesh(core_axis_name="c", subcore_axis_name="s")

    xs, ws = pl.kernel(
        _sc_dispatch_body,
        out_type=[
            jax.ShapeDtypeStruct((NROWS, D), jnp.float32),
            jax.ShapeDtypeStruct((NROWS, 128), jnp.float32),
        ],
        mesh=mesh,
        scratch_types=[
            pltpu.VMEM((CH, D), jnp.float32),
            pltpu.VMEM((CH, D), jnp.float32),
            pltpu.VMEM((PW,), jnp.int32),
            pltpu.VMEM((PW,), jnp.int32),
            pltpu.VMEM((PW, 128), jnp.float32),
            pltpu.VMEM((PW, 128), jnp.float32),
            pltpu.SemaphoreType.DMA,
            pltpu.SemaphoreType.DMA,
        ],
    )(x, p0f, p1f, w0, w1c)

    ys = pl.pallas_call(
        _moe_body,
        grid_spec=pltpu.PrefetchScalarGridSpec(
            num_scalar_prefetch=1,
            grid=(NBLK,),
            in_specs=[
                pl.BlockSpec((BTM, D), lambda b, m: (b, 0)),
                pl.BlockSpec((1, FF, D), lambda b, m: (m[b], 0, 0)),
                pl.BlockSpec((1, FF, D), lambda b, m: (m[b], 0, 0)),
                pl.BlockSpec((1, D, FF), lambda b, m: (m[b], 0, 0)),
                pl.BlockSpec((BTM, 128), lambda b, m: (b, 0)),
            ],
            out_specs=pl.BlockSpec((BTM, D), lambda b, m: (b, 0)),
        ),
        out_shape=jax.ShapeDtypeStruct((NROWS, D), jnp.float32),
        compiler_params=pltpu.CompilerParams(
            dimension_semantics=("arbitrary",)),
    )(bmap.reshape(NBLKP), xs, w1, w3, w2, ws)

    DH = D // 2
    ys2 = ys.reshape(2 * NROWS, DH)
    two = jnp.arange(2, dtype=jnp.int32).reshape(1, 2)
    p0d = (2 * p0 + two).reshape(2 * T)
    p1d = (2 * p1 + two).reshape(2 * T)
    PW2 = 2 * PW
    out2 = pl.kernel(
        _sc_combine_body,
        out_type=jax.ShapeDtypeStruct((2 * T, DH), jnp.float32),
        mesh=mesh,
        scratch_types=[
            pltpu.VMEM((16, DH), jnp.float32),
            pltpu.VMEM((16, DH), jnp.float32),
            pltpu.VMEM((16, DH), jnp.float32),
            pltpu.VMEM((16, DH), jnp.float32),
            pltpu.VMEM((16, DH), jnp.float32),
            pltpu.VMEM((16, DH), jnp.float32),
            pltpu.VMEM((PW2,), jnp.int32),
            pltpu.VMEM((PW2,), jnp.int32),
            pltpu.SemaphoreType.DMA,
            pltpu.SemaphoreType.DMA,
        ],
    )(ys2, p0d, p1d)
    return out2.reshape(T, D)


# speedup vs baseline: 1.2545x; 1.2545x over previous
import functools

import jax
import jax.numpy as jnp
from jax import lax
from jax.experimental import pallas as pl
from jax.experimental.pallas import tpu as pltpu
from jax.experimental.pallas import tpu_sc as plsc

BTM = 256
CH = 16


def _routing_body(x_ref, gw_ref, w0_ref, w1_ref, p0_ref, p1_ref, bmap_ref,
                  cnt_ref, e0s_ref, e1s_ref, r0s_ref, r1s_ref):
    s = pl.program_id(0)
    NS = pl.num_programs(0)
    S = x_ref.shape[0]
    E = gw_ref.shape[0]
    NBLKP = bmap_ref.shape[1]
    T = e0s_ref.shape[0]

    @pl.when(s == 0)
    def _():
        cnt_ref[...] = jnp.zeros_like(cnt_ref)

    x = x_ref[...]
    logits = lax.dot_general(x, gw_ref[...], (((1,), (1,)), ((), ())),
                             preferred_element_type=jnp.float32)
    iota = lax.broadcasted_iota(jnp.int32, (S, E), 1)
    m = jnp.max(logits, axis=-1, keepdims=True)
    p = jnp.exp(logits - m)
    m1 = jnp.max(p, axis=-1, keepdims=True)
    i1 = jnp.min(jnp.where(p == m1, iota, E), axis=-1, keepdims=True)
    mask1 = iota == i1
    p2 = jnp.where(mask1, -jnp.inf, p)
    m2 = jnp.max(p2, axis=-1, keepdims=True)
    i2 = jnp.min(jnp.where(p2 == m2, iota, E), axis=-1, keepdims=True)
    mask2 = iota == i2
    denom = m1 + m2
    w0_ref[...] = m1 / denom
    w1_ref[...] = m2 / denom

    oh = mask1.astype(jnp.float32) + mask2.astype(jnp.float32)
    tril = (lax.broadcasted_iota(jnp.int32, (S, S), 0) >
            lax.broadcasted_iota(jnp.int32, (S, S), 1)).astype(jnp.float32)
    csum = lax.dot_general(tril, oh, (((1,), (0,)), ((), ())),
                           preferred_element_type=jnp.float32)
    csum = csum + cnt_ref[...].astype(jnp.float32)
    r0 = jnp.sum(jnp.where(mask1, csum, 0.0), axis=-1,
                 keepdims=True).astype(jnp.int32)
    r1 = jnp.sum(jnp.where(mask2, csum, 0.0), axis=-1,
                 keepdims=True).astype(jnp.int32)

    sl = pl.ds(s * S, S)
    e0s_ref[sl] = i1
    e1s_ref[sl] = i2
    r0s_ref[sl] = r0
    r1s_ref[sl] = r1
    cnt_ref[...] = (cnt_ref[...] +
                    jnp.sum(oh, axis=0, keepdims=True).astype(jnp.int32))

    @pl.when(s == NS - 1)
    def _():
        cnt = cnt_ref[...]
        padded = ((cnt + (BTM - 1)) // BTM) * BTM
        upper = (lax.broadcasted_iota(jnp.int32, (E, E), 0) <
                 lax.broadcasted_iota(jnp.int32, (E, E), 1)).astype(jnp.float32)
        offs = lax.dot_general(padded.astype(jnp.float32), upper,
                               (((1,), (0,)), ((), ())),
                               preferred_element_type=jnp.float32)
        iota_te = lax.broadcasted_iota(jnp.int32, (T, E), 1)
        mask0 = (iota_te == e0s_ref[...]).astype(jnp.float32)
        mask1a = (iota_te == e1s_ref[...]).astype(jnp.float32)
        base0 = lax.dot_general(mask0, offs, (((1,), (1,)), ((), ())),
                                preferred_element_type=jnp.float32)
        base1 = lax.dot_general(mask1a, offs, (((1,), (1,)), ((), ())),
                                preferred_element_type=jnp.float32)
        p0_ref[...] = base0.astype(jnp.int32) + r0s_ref[...]
        p1_ref[...] = base1.astype(jnp.int32) + r1s_ref[...]

        incl = offs + padded.astype(jnp.float32)
        total = jnp.sum(padded)
        lane = lax.broadcasted_iota(jnp.int32, (1, NBLKP), 1)
        pos = jnp.minimum(lane * BTM, total - BTM).astype(jnp.float32)
        bexp = jnp.zeros((1, NBLKP), jnp.int32)
        for e in range(E):
            bexp = bexp + (pos >= incl[:, e:e + 1]).astype(jnp.int32)
        nblk_used = (total // BTM).astype(jnp.int32)
        bmap_ref[...] = jnp.where(lane == NBLKP - 1, nblk_used, bexp)


def _moe_body(bmap_ref, xs_ref, w1_ref, w3_ref, w2_ref, o_ref):
    b = pl.program_id(0)
    nused = bmap_ref[bmap_ref.shape[0] - 1]

    @pl.when(b < nused)
    def _():
        x = xs_ref[...]
        g = lax.dot_general(x, w1_ref[0], (((1,), (1,)), ((), ())),
                            preferred_element_type=jnp.float32)
        u = lax.dot_general(x, w3_ref[0], (((1,), (1,)), ((), ())),
                            preferred_element_type=jnp.float32)
        h = g * lax.logistic(g) * u
        o_ref[...] = lax.dot_general(h, w2_ref[0], (((1,), (1,)), ((), ())),
                                     preferred_element_type=jnp.float32)


def _combine_body(y0_ref, y1_ref, w0_ref, w1_ref, o_ref):
    o_ref[...] = w0_ref[...] * y0_ref[...] + w1_ref[...] * y1_ref[...]


def _sc_worker_id():
    return lax.axis_index("s") * 2 + lax.axis_index("c")


def _sc_dispatch_body(x_hbm, p0_hbm, p1_hbm, xs_hbm, rows_a, rows_b,
                      p0_v, p1_v, lsem, ssem):
    T = x_hbm.shape[0]
    wid = _sc_worker_id()
    per_w = T // 32
    nch = per_w // CH
    base = wid * per_w
    pltpu.sync_copy(p0_hbm.at[pl.ds(base, per_w)], p0_v)
    pltpu.sync_copy(p1_hbm.at[pl.ds(base, per_w)], p1_v)
    bufs = [rows_a, rows_b]
    loads = {0: pltpu.async_copy(x_hbm.at[pl.ds(base, CH)], rows_a, lsem)}
    scats = {}
    for ci in range(nch):
        if ci >= 1:
            scats[ci - 1][0].wait()
            scats[ci - 1][1].wait()
        if ci + 1 < nch:
            loads[ci + 1] = pltpu.async_copy(
                x_hbm.at[pl.ds(base + (ci + 1) * CH, CH)],
                bufs[(ci + 1) % 2], lsem)
        loads[ci].wait()
        sl = pl.ds(ci * CH, CH)
        i0 = p0_v[sl]
        i1 = p1_v[sl]
        buf = bufs[ci % 2]
        scats[ci] = (pltpu.async_copy(buf, xs_hbm.at[i0], ssem),
                     pltpu.async_copy(buf, xs_hbm.at[i1], ssem))
    scats[nch - 1][0].wait()
    scats[nch - 1][1].wait()


def _sc_combine_body(ys_hbm, p0_hbm, p1_hbm, y0_hbm, y1_hbm, rows_a, rows_b,
                     p0_v, p1_v, g0sem, g1sem, s0sem, s1sem):
    T = y0_hbm.shape[0]
    wid = _sc_worker_id()
    per_w = T // 32
    nch = per_w // CH
    base = wid * per_w
    pltpu.sync_copy(p0_hbm.at[pl.ds(base, per_w)], p0_v)
    pltpu.sync_copy(p1_hbm.at[pl.ds(base, per_w)], p1_v)

    g0 = pltpu.async_copy(ys_hbm.at[p0_v[pl.ds(0, CH)]], rows_a, g0sem)
    s1 = None
    for ci in range(nch):
        dst = pl.ds(base + ci * CH, CH)
        g0.wait()
        s0 = pltpu.async_copy(rows_a, y0_hbm.at[dst], s0sem)
        if s1 is not None:
            s1.wait()
        g1 = pltpu.async_copy(ys_hbm.at[p1_v[pl.ds(ci * CH, CH)]],
                              rows_b, g1sem)
        s0.wait()
        if ci + 1 < nch:
            g0 = pltpu.async_copy(ys_hbm.at[p0_v[pl.ds((ci + 1) * CH, CH)]],
                                  rows_a, g0sem)
        g1.wait()
        s1 = pltpu.async_copy(rows_b, y1_hbm.at[dst], s1sem)
    s1.wait()


def kernel(hidden_states, gate_w, w1, w2, w3, num_global_tokens,
           max_num_tokens_per_gpu):
    T, D = hidden_states.shape
    E, FF, _ = w1.shape
    K = 2
    x = hidden_states.astype(jnp.float32)

    NBLK = (T * K) // BTM + E
    NBLKP = NBLK + 1
    NROWS = NBLK * BTM
    S = T // 2

    w0, w1c, p0, p1, bmap = pl.pallas_call(
        _routing_body,
        grid=(T // S,),
        in_specs=[
            pl.BlockSpec((S, D), lambda s: (s, 0)),
            pl.BlockSpec((E, D), lambda s: (0, 0)),
        ],
        out_specs=[
            pl.BlockSpec((S, 1), lambda s: (s, 0)),
            pl.BlockSpec((S, 1), lambda s: (s, 0)),
            pl.BlockSpec((T, 1), lambda s: (0, 0)),
            pl.BlockSpec((T, 1), lambda s: (0, 0)),
            pl.BlockSpec((1, NBLKP), lambda s: (0, 0)),
        ],
        out_shape=[
            jax.ShapeDtypeStruct((T, 1), jnp.float32),
            jax.ShapeDtypeStruct((T, 1), jnp.float32),
            jax.ShapeDtypeStruct((T, 1), jnp.int32),
            jax.ShapeDtypeStruct((T, 1), jnp.int32),
            jax.ShapeDtypeStruct((1, NBLKP), jnp.int32),
        ],
        scratch_shapes=[
            pltpu.VMEM((1, E), jnp.int32),
            pltpu.VMEM((T, 1), jnp.int32),
            pltpu.VMEM((T, 1), jnp.int32),
            pltpu.VMEM((T, 1), jnp.int32),
            pltpu.VMEM((T, 1), jnp.int32),
        ],
        compiler_params=pltpu.CompilerParams(
            dimension_semantics=("arbitrary",)),
    )(x, gate_w)

    p0f = p0.reshape(T)
    p1f = p1.reshape(T)
    PW = T // 32

    mesh = plsc.VectorSubcoreMesh(core_axis_name="c", subcore_axis_name="s")

    xs = pl.kernel(
        _sc_dispatch_body,
        out_type=jax.ShapeDtypeStruct((NROWS, D), jnp.float32),
        mesh=mesh,
        scratch_types=[
            pltpu.VMEM((CH, D), jnp.float32),
            pltpu.VMEM((CH, D), jnp.float32),
            pltpu.VMEM((PW,), jnp.int32),
            pltpu.VMEM((PW,), jnp.int32),
            pltpu.SemaphoreType.DMA,
            pltpu.SemaphoreType.DMA,
        ],
    )(x, p0f, p1f)

    ys = pl.pallas_call(
        _moe_body,
        grid_spec=pltpu.PrefetchScalarGridSpec(
            num_scalar_prefetch=1,
            grid=(NBLK,),
            in_specs=[
                pl.BlockSpec((BTM, D), lambda b, m: (b, 0)),
                pl.BlockSpec((1, FF, D), lambda b, m: (m[b], 0, 0)),
                pl.BlockSpec((1, FF, D), lambda b, m: (m[b], 0, 0)),
                pl.BlockSpec((1, D, FF), lambda b, m: (m[b], 0, 0)),
            ],
            out_specs=pl.BlockSpec((BTM, D), lambda b, m: (b, 0)),
        ),
        out_shape=jax.ShapeDtypeStruct((NROWS, D), jnp.float32),
        compiler_params=pltpu.CompilerParams(
            dimension_semantics=("arbitrary",)),
    )(bmap.reshape(NBLKP), xs, w1, w3, w2)

    y0, y1 = pl.kernel(
        _sc_combine_body,
        out_type=[
            jax.ShapeDtypeStruct((T, D), jnp.float32),
            jax.ShapeDtypeStruct((T, D), jnp.float32),
        ],
        mesh=mesh,
        scratch_types=[
            pltpu.VMEM((CH, D), jnp.float32),
            pltpu.VMEM((CH, D), jnp.float32),
            pltpu.VMEM((PW,), jnp.int32),
            pltpu.VMEM((PW,), jnp.int32),
            pltpu.SemaphoreType.DMA,
            pltpu.SemaphoreType.DMA,
            pltpu.SemaphoreType.DMA,
            pltpu.SemaphoreType.DMA,
        ],
    )(ys, p0f, p1f)

    BTC = 512
    out = pl.pallas_call(
        _combine_body,
        grid=(T // BTC,),
        in_specs=[
            pl.BlockSpec((BTC, D), lambda t: (t, 0)),
            pl.BlockSpec((BTC, D), lambda t: (t, 0)),
            pl.BlockSpec((BTC, 1), lambda t: (t, 0)),
            pl.BlockSpec((BTC, 1), lambda t: (t, 0)),
        ],
        out_specs=pl.BlockSpec((BTC, D), lambda t: (t, 0)),
        out_shape=jax.ShapeDtypeStruct((T, D), jnp.float32),
    )(y0, y1, w0, w1c)
    return out
